# Initial kernel scaffold; baseline (speedup 1.0000x reference)
import jax
import jax.numpy as jnp
from jax.experimental import pallas as pl


def kernel(x, edge_index, W1, b1, W2, b2):
    raise NotImplementedError("write your pallas kernel here")


# speedup vs baseline: 9.2568x; 9.2568x over previous
import functools

import jax
import jax.numpy as jnp
from jax import lax
from jax.experimental import pallas as pl
from jax.experimental.pallas import tpu as pltpu
from jax.experimental.pallas import tpu_sc as plsc

N = 10000
R = 12800
ROW_BLK = 200
NC = 2
NS = 16
B = 128
D = 128


def _make_deg(e_pad):
  per_tile = e_pad // (NC * NS)
  batches = per_tile // B
  rps = R // NS
  mesh = plsc.VectorSubcoreMesh(core_axis_name="c", subcore_axis_name="s")

  @functools.partial(
      pl.kernel, mesh=mesh,
      out_type=jax.ShapeDtypeStruct((NC * R, D), jnp.float32),
      scratch_types=[
          pltpu.VMEM((B,), jnp.int32),
          pltpu.VMEM((B, D), jnp.float32),
          pltpu.VMEM_SHARED((R, D), jnp.float32),
      ],
  )
  def deg_kernel(dst_hbm, zero_hbm, ones_hbm, out_hbm, dst_v, ones_v, acc_sh):
    c = lax.axis_index("c")
    s = lax.axis_index("s")
    wid = s * NC + c
    r0 = s * rps
    pltpu.sync_copy(ones_hbm, ones_v)
    pltpu.sync_copy(zero_hbm.at[pl.ds(r0, rps)], acc_sh.at[pl.ds(r0, rps)])
    plsc.subcore_barrier()
    base = wid * per_tile

    def body(i, carry):
      pltpu.sync_copy(dst_hbm.at[pl.ds(base + i * B, B)], dst_v)
      pltpu.sync_copy(ones_v, acc_sh.at[dst_v], add=True)
      return carry

    lax.fori_loop(0, batches, body, 0)
    plsc.subcore_barrier()
    pltpu.sync_copy(acc_sh.at[pl.ds(r0, rps)],
                    out_hbm.at[pl.ds(c * R + r0, rps)])

  return deg_kernel


def _make_agg(e_pad):
  per_tile = e_pad // (NC * NS)
  batches = per_tile // B
  rps = R // NS
  mesh = plsc.VectorSubcoreMesh(core_axis_name="c", subcore_axis_name="s")

  @functools.partial(
      pl.kernel, mesh=mesh,
      out_type=jax.ShapeDtypeStruct((NC * R, D), jnp.float32),
      scratch_types=[
          pltpu.VMEM((B,), jnp.int32),
          pltpu.VMEM((B,), jnp.int32),
          pltpu.VMEM((B, D), jnp.float32),
          pltpu.VMEM_SHARED((R, D), jnp.float32),
          pltpu.SemaphoreType.DMA,
      ],
  )
  def agg_kernel(g_hbm, src_hbm, dst_hbm, zero_hbm, out_hbm,
                 src_v, dst_v, rows_v, acc_sh, sem):
    c = lax.axis_index("c")
    s = lax.axis_index("s")
    wid = s * NC + c
    r0 = s * rps
    pltpu.sync_copy(zero_hbm.at[pl.ds(r0, rps)], acc_sh.at[pl.ds(r0, rps)])
    plsc.subcore_barrier()
    base = wid * per_tile

    def body(i, carry):
      off = base + i * B
      pltpu.sync_copy(src_hbm.at[pl.ds(off, B)], src_v)
      pltpu.sync_copy(dst_hbm.at[pl.ds(off, B)], dst_v)
      pltpu.async_copy(g_hbm.at[src_v], rows_v, sem).wait()
      pltpu.sync_copy(rows_v, acc_sh.at[dst_v], add=True)
      return carry

    lax.fori_loop(0, batches, body, 0)
    plsc.subcore_barrier()
    pltpu.sync_copy(acc_sh.at[pl.ds(r0, rps)],
                    out_hbm.at[pl.ds(c * R + r0, rps)])

  return agg_kernel


def _tc1_body(da_ref, db_ref, x_ref, w1_ref, g1_ref, dinv_ref):
  deg = da_ref[:, 0:1] + db_ref[:, 0:1] + 1.0
  dinv = lax.rsqrt(deg)
  h = jnp.dot(x_ref[...], w1_ref[...], preferred_element_type=jnp.float32)
  g1_ref[...] = h * dinv
  dinv_ref[...] = jnp.broadcast_to(dinv, (ROW_BLK, 128))


def _tc2_body(aa_ref, ab_ref, g1_ref, dinv_ref, b1_ref, w2_ref, g2_ref):
  dinv = dinv_ref[:, 0:1]
  s = dinv * (aa_ref[...] + ab_ref[...] + g1_ref[...]) + b1_ref[...]
  o = jnp.maximum(s, 0.0)
  h2 = jnp.dot(o, w2_ref[...], preferred_element_type=jnp.float32)
  g2_ref[...] = jnp.concatenate(
      [h2 * dinv, jnp.zeros((ROW_BLK, 64), jnp.float32)], axis=1)


def _tc3_body(aa_ref, ab_ref, g2_ref, dinv_ref, b2_ref, out_ref):
  dinv = dinv_ref[:, 0:1]
  z = dinv * (aa_ref[:, :64] + ab_ref[:, :64] + g2_ref[:, :64]) + b2_ref[...]
  m = jnp.max(z, axis=1, keepdims=True)
  lse = jnp.log(jnp.sum(jnp.exp(z - m), axis=1, keepdims=True)) + m
  out_ref[...] = z - lse


def _row_spec(d, off_blocks=0):
  return pl.BlockSpec((ROW_BLK, d), lambda i, o=off_blocks: (i + o, 0))


def _full_spec(r, c):
  return pl.BlockSpec((r, c), lambda i: (0, 0))


def kernel(x, edge_index, W1, b1, W2, b2):
  e = edge_index.shape[1]
  e_pad = -(-e // (NC * NS * B)) * (NC * NS * B)
  src = edge_index[0].astype(jnp.int32)
  dst = edge_index[1].astype(jnp.int32)
  pad = e_pad - e
  src_p = jnp.concatenate([src, jnp.zeros((pad,), jnp.int32)])
  dst_p = jnp.concatenate([dst, jnp.full((pad,), N, jnp.int32)])

  z128 = jnp.zeros((R, D), jnp.float32)
  ones128 = jnp.ones((B, D), jnp.float32)

  deg_p = _make_deg(e_pad)(dst_p, z128, ones128)

  grid = (N // ROW_BLK,)
  off = R // ROW_BLK
  g1, dinv = pl.pallas_call(
      _tc1_body,
      grid=grid,
      in_specs=[_row_spec(D), _row_spec(D, off),
                _row_spec(128), _full_spec(128, 128)],
      out_specs=[_row_spec(128), _row_spec(128)],
      out_shape=[jax.ShapeDtypeStruct((N, 128), jnp.float32),
                 jax.ShapeDtypeStruct((N, 128), jnp.float32)],
  )(deg_p, deg_p, x, W1)

  acc1 = _make_agg(e_pad)(g1, src_p, dst_p, z128)

  g2 = pl.pallas_call(
      _tc2_body,
      grid=grid,
      in_specs=[_row_spec(D), _row_spec(D, off), _row_spec(128),
                _row_spec(128), _full_spec(1, 128), _full_spec(128, 64)],
      out_specs=_row_spec(128),
      out_shape=jax.ShapeDtypeStruct((N, 128), jnp.float32),
  )(acc1, acc1, g1, dinv, b1.reshape(1, 128), W2)

  acc2 = _make_agg(e_pad)(g2, src_p, dst_p, z128)

  out = pl.pallas_call(
      _tc3_body,
      grid=grid,
      in_specs=[_row_spec(D), _row_spec(D, off), _row_spec(128),
                _row_spec(128), _full_spec(1, 64)],
      out_specs=_row_spec(64),
      out_shape=jax.ShapeDtypeStruct((N, 64), jnp.float32),
  )(acc2, acc2, g2, dinv, b2.reshape(1, 64))

  return out
